# baseline (speedup 1.0000x reference)
import jax
import jax.numpy as jnp
from jax import lax
from jax.experimental import pallas as pl
from jax.experimental.pallas import tpu as pltpu
from jax.experimental.pallas import tpu_sc as plsc

N = 10000
E = 320000
D = 128

NC = 2
NS = 16
NW = NC * NS
CH = 128
KB = 8
NDEP = 2
EPW = 10240
K = EPW // CH
E_PAD = NW * EPW
NBLK = E_PAD // (KB * CH)
B0 = 15
B1 = 5
NPAD = 10240
RPT = NPAD // NS
DUMMY = N
LW = 128
NR = NPAD // LW
REG = 8

_mesh = plsc.VectorSubcoreMesh(core_axis_name="c", subcore_axis_name="s",
                               num_cores=NC, num_subcores=NS)


def _segsum_body(x_hbm, src_hbm, dst_hbm, agg_out, src_v, dst_v, b0, b1,
                 acc_sh, g0, g1, s0, s1):
  c = lax.axis_index("c")
  s = lax.axis_index("s")

  zv = jnp.zeros((16,), jnp.float32)

  def zrow(i, carry):
    for kk in range(D // 16):
      b0[i, pl.ds(kk * 16, 16)] = zv
    return carry

  lax.fori_loop(0, CH, zrow, 0)
  for r in range(RPT // CH):
    pltpu.sync_copy(b0, acc_sh.at[pl.ds(s * RPT + r * CH, CH)])
  plsc.subcore_barrier()

  bufs = (b0, b1)
  gsems = (g0, g1)
  ssems = (s0, s1)
  nblk = B0 + c * (B1 - B0)
  base = (1 - c) * (s * B0) + c * (NS * B0 + s * B1)

  def blk(b, carry):
    pltpu.sync_copy(src_hbm.at[base + b], src_v)
    pltpu.sync_copy(dst_hbm.at[base + b], dst_v)
    gcps = [None] * KB
    scps = [None] * KB
    for t in range(KB + NDEP - 1):
      jj = t
      if jj < KB:
        slot = jj % NDEP
        if jj >= NDEP:
          scps[jj - NDEP].wait()
        gcps[jj] = pltpu.async_copy(x_hbm.at[src_v.at[jj]], bufs[slot],
                                    gsems[slot])
      j2 = t - (NDEP - 1)
      if 0 <= j2 < KB:
        gcps[j2].wait()
        scps[j2] = pltpu.async_copy(bufs[j2 % NDEP],
                                    acc_sh.at[dst_v.at[j2]],
                                    ssems[j2 % NDEP], add=True)
    for j2 in range(KB - NDEP, KB):
      scps[j2].wait()
    return carry

  lax.fori_loop(0, nblk, blk, 0)
  plsc.subcore_barrier()

  pltpu.sync_copy(acc_sh.at[pl.ds(s * RPT, RPT)],
                  agg_out.at[c, pl.ds(s * RPT, RPT)])


_segsum = pl.kernel(
    _segsum_body,
    out_type=[jax.ShapeDtypeStruct((NC, NPAD, D), jnp.float32)],
    mesh=_mesh,
    scratch_types=[
        pltpu.VMEM((KB, CH), jnp.int32),
        pltpu.VMEM((KB, CH), jnp.int32),
        pltpu.VMEM((CH, D), jnp.float32),
        pltpu.VMEM((CH, D), jnp.float32),
        pltpu.VMEM_SHARED((NPAD, D), jnp.float32),
        pltpu.SemaphoreType.DMA,
        pltpu.SemaphoreType.DMA,
        pltpu.SemaphoreType.DMA,
        pltpu.SemaphoreType.DMA,
    ])


def _count_body(dst_hbm, cnt_out, dst_v, hist):
  c = lax.axis_index("c")
  s = lax.axis_index("s")
  w = s * NC + c
  pltpu.sync_copy(dst_hbm.at[w], dst_v)
  zv = jnp.zeros((16,), jnp.float32)

  def zrow(i, carry):
    for kk in range(LW // 16):
      hist[i, pl.ds(kk * 16, 16)] = zv
    return carry

  lax.fori_loop(0, REG * NR, zrow, 0)

  lane = lax.iota(jnp.int32, 16)
  region = jnp.bitwise_and(lane, REG - 1)
  mlo = lane < 8
  mhi = lane >= 8
  onesv = jnp.ones((16,), jnp.float32)

  def jrow(j, carry):
    for kk in range(CH // 16):
      d = dst_v[j, pl.ds(kk * 16, 16)]
      row = region * NR + lax.shift_right_logical(d, 7)
      col = jnp.bitwise_and(d, 127)
      plsc.addupdate_scatter(hist, [row, col], onesv, mask=mlo)
      plsc.addupdate_scatter(hist, [row, col], onesv, mask=mhi)
    return carry

  lax.fori_loop(0, K, jrow, 0)

  def rrow(i, carry):
    for kk in range(LW // 16):
      acc = hist[i, pl.ds(kk * 16, 16)]
      for r in range(1, REG):
        acc = acc + hist[r * NR + i, pl.ds(kk * 16, 16)]
      hist[i, pl.ds(kk * 16, 16)] = acc
    return carry

  lax.fori_loop(0, NR, rrow, 0)
  pltpu.sync_copy(hist.at[pl.ds(0, NR)], cnt_out.at[w])


_count = pl.kernel(
    _count_body,
    out_type=jax.ShapeDtypeStruct((NW, NR, LW), jnp.float32),
    mesh=_mesh,
    scratch_types=[pltpu.VMEM((K, CH), jnp.int32),
                   pltpu.VMEM((REG * NR, LW), jnp.float32)],
    compiler_params=pltpu.CompilerParams(needs_layout_passes=False))


def _cntsum_body(cnt_ref, out_ref):
  out_ref[...] = jnp.maximum(jnp.sum(cnt_ref[...], axis=0), 1.0)


_cntsum = pl.pallas_call(
    _cntsum_body,
    out_shape=jax.ShapeDtypeStruct((NR, LW), jnp.float32))


def _dense_body_prelu(agg_ref, cnt_ref, x_ref, wl_ref, b_ref, wr_ref,
                      a_ref, out_ref):
  mean = (agg_ref[0, :N, :] + agg_ref[1, :N, :]) / cnt_ref[...]
  h = lax.dot_general(mean, wl_ref[...], (((1,), (1,)), ((), ())),
                      preferred_element_type=jnp.float32)
  h = h + b_ref[...]
  h = h + lax.dot_general(x_ref[...], wr_ref[...], (((1,), (1,)), ((), ())),
                          preferred_element_type=jnp.float32)
  a = a_ref[...]
  out_ref[...] = jnp.maximum(h, 0.0) + a * jnp.minimum(h, 0.0)


def _dense_body(agg_ref, cnt_ref, x_ref, wl_ref, b_ref, wr_ref, out_ref):
  mean = (agg_ref[0, :N, :] + agg_ref[1, :N, :]) / cnt_ref[...]
  h = lax.dot_general(mean, wl_ref[...], (((1,), (1,)), ((), ())),
                      preferred_element_type=jnp.float32)
  h = h + b_ref[...]
  h = h + lax.dot_general(x_ref[...], wr_ref[...], (((1,), (1,)), ((), ())),
                          preferred_element_type=jnp.float32)
  out_ref[...] = h


_dense1 = pl.pallas_call(
    _dense_body_prelu,
    out_shape=jax.ShapeDtypeStruct((N, D), jnp.float32))
_dense2 = pl.pallas_call(
    _dense_body,
    out_shape=jax.ShapeDtypeStruct((N, D), jnp.float32))


@jax.jit
def kernel(x, edge_index, W1l, b1, W1r, a1, W2l, b2, W2r):
  src = edge_index[0]
  dst = edge_index[1]
  pad = E_PAD - E
  srcf = jnp.concatenate([src, jnp.zeros((pad,), jnp.int32)])
  dpad = DUMMY + jnp.arange(pad, dtype=jnp.int32) % (NPAD - N)
  dstf = jnp.concatenate([dst, dpad])
  srcp = srcf.reshape(NBLK, KB, CH)
  srcp = srcp + (jnp.arange(NBLK, dtype=jnp.int32) % 2 * N)[:, None, None]
  dstp = dstf.reshape(NBLK, KB, CH)

  cnt32 = _count(dstf.reshape(NW, K, CH))
  cntc = _cntsum(cnt32).reshape(NPAD, 1)[:N]
  b1r = b1.reshape(1, D)
  b2r = b2.reshape(1, D)

  x2 = jnp.concatenate([x, x])
  (aggp,) = _segsum(x2, srcp, dstp)
  h = _dense1(aggp, cntc, x, W1l, b1r, W1r, a1.reshape(1, 1))
  h2 = jnp.concatenate([h, h])
  (aggp2,) = _segsum(h2, srcp, dstp)
  out = _dense2(aggp2, cntc, h, W2l, b2r, W2r)
  return out


# speedup vs baseline: 1.0596x; 1.0596x over previous
import jax
import jax.numpy as jnp
from jax import lax
from jax.experimental import pallas as pl
from jax.experimental.pallas import tpu as pltpu
from jax.experimental.pallas import tpu_sc as plsc

N = 10000
E = 320000
D = 128

NC = 2
NS = 16
NW = NC * NS
CH = 128
KB = 8
NDEP = 2
EPW = 10240
K = EPW // CH
E_PAD = NW * EPW
NBLK = E_PAD // (KB * CH)
B0 = 16
B1 = 4
NPAD = 10240
RPT = NPAD // NS
DUMMY = N
LW = 128
NR = NPAD // LW
REG = 8

_mesh = plsc.VectorSubcoreMesh(core_axis_name="c", subcore_axis_name="s",
                               num_cores=NC, num_subcores=NS)


def _segsum_body(x_hbm, src_hbm, dst_hbm, agg_out, src_v, dst_v, b0, b1,
                 acc_sh, g0, g1, s0, s1):
  c = lax.axis_index("c")
  s = lax.axis_index("s")

  zv = jnp.zeros((16,), jnp.float32)

  def zrow(i, carry):
    for kk in range(D // 16):
      b0[i, pl.ds(kk * 16, 16)] = zv
    return carry

  lax.fori_loop(0, CH, zrow, 0)
  for r in range(RPT // CH):
    pltpu.sync_copy(b0, acc_sh.at[pl.ds(s * RPT + r * CH, CH)])
  plsc.subcore_barrier()

  bufs = (b0, b1)
  gsems = (g0, g1)
  ssems = (s0, s1)
  nblk = B0 + c * (B1 - B0)
  base = (1 - c) * (s * B0) + c * (NS * B0 + s * B1)

  def blk(b, carry):
    pltpu.sync_copy(src_hbm.at[base + b], src_v)
    pltpu.sync_copy(dst_hbm.at[base + b], dst_v)
    gcps = [None] * KB
    scps = [None] * KB
    for t in range(KB + NDEP - 1):
      jj = t
      if jj < KB:
        slot = jj % NDEP
        if jj >= NDEP:
          scps[jj - NDEP].wait()
        gcps[jj] = pltpu.async_copy(x_hbm.at[src_v.at[jj]], bufs[slot],
                                    gsems[slot])
      j2 = t - (NDEP - 1)
      if 0 <= j2 < KB:
        gcps[j2].wait()
        scps[j2] = pltpu.async_copy(bufs[j2 % NDEP],
                                    acc_sh.at[dst_v.at[j2]],
                                    ssems[j2 % NDEP], add=True)
    for j2 in range(KB - NDEP, KB):
      scps[j2].wait()
    return carry

  lax.fori_loop(0, nblk, blk, 0)
  plsc.subcore_barrier()

  pltpu.sync_copy(acc_sh.at[pl.ds(s * RPT, RPT)],
                  agg_out.at[c, pl.ds(s * RPT, RPT)])


_segsum = pl.kernel(
    _segsum_body,
    out_type=[jax.ShapeDtypeStruct((NC, NPAD, D), jnp.float32)],
    mesh=_mesh,
    scratch_types=[
        pltpu.VMEM((KB, CH), jnp.int32),
        pltpu.VMEM((KB, CH), jnp.int32),
        pltpu.VMEM((CH, D), jnp.float32),
        pltpu.VMEM((CH, D), jnp.float32),
        pltpu.VMEM_SHARED((NPAD, D), jnp.float32),
        pltpu.SemaphoreType.DMA,
        pltpu.SemaphoreType.DMA,
        pltpu.SemaphoreType.DMA,
        pltpu.SemaphoreType.DMA,
    ])


def _count_body(dst_hbm, cnt_out, dst_v, hist):
  c = lax.axis_index("c")
  s = lax.axis_index("s")
  w = s * NC + c
  pltpu.sync_copy(dst_hbm.at[w], dst_v)
  zv = jnp.zeros((16,), jnp.float32)

  def zrow(i, carry):
    for kk in range(LW // 16):
      hist[i, pl.ds(kk * 16, 16)] = zv
    return carry

  lax.fori_loop(0, REG * NR, zrow, 0)

  lane = lax.iota(jnp.int32, 16)
  region = jnp.bitwise_and(lane, REG - 1)
  mlo = lane < 8
  mhi = lane >= 8
  onesv = jnp.ones((16,), jnp.float32)

  def jrow(j, carry):
    for kk in range(CH // 16):
      d = dst_v[j, pl.ds(kk * 16, 16)]
      row = region * NR + lax.shift_right_logical(d, 7)
      col = jnp.bitwise_and(d, 127)
      plsc.addupdate_scatter(hist, [row, col], onesv, mask=mlo)
      plsc.addupdate_scatter(hist, [row, col], onesv, mask=mhi)
    return carry

  lax.fori_loop(0, K, jrow, 0)

  def rrow(i, carry):
    for kk in range(LW // 16):
      acc = hist[i, pl.ds(kk * 16, 16)]
      for r in range(1, REG):
        acc = acc + hist[r * NR + i, pl.ds(kk * 16, 16)]
      hist[i, pl.ds(kk * 16, 16)] = acc
    return carry

  lax.fori_loop(0, NR, rrow, 0)
  pltpu.sync_copy(hist.at[pl.ds(0, NR)], cnt_out.at[w])


_count = pl.kernel(
    _count_body,
    out_type=jax.ShapeDtypeStruct((NW, NR, LW), jnp.float32),
    mesh=_mesh,
    scratch_types=[pltpu.VMEM((K, CH), jnp.int32),
                   pltpu.VMEM((REG * NR, LW), jnp.float32)],
    compiler_params=pltpu.CompilerParams(needs_layout_passes=False))


def _cntsum_body(cnt_ref, out_ref):
  out_ref[...] = jnp.maximum(jnp.sum(cnt_ref[...], axis=0), 1.0)


_cntsum = pl.pallas_call(
    _cntsum_body,
    out_shape=jax.ShapeDtypeStruct((NR, LW), jnp.float32))


def _dense_body_prelu(agg_ref, cnt_ref, x_ref, wl_ref, b_ref, wr_ref,
                      a_ref, out_ref):
  mean = (agg_ref[0, :N, :] + agg_ref[1, :N, :]) / cnt_ref[...]
  h = lax.dot_general(mean, wl_ref[...], (((1,), (1,)), ((), ())),
                      preferred_element_type=jnp.float32)
  h = h + b_ref[...]
  h = h + lax.dot_general(x_ref[...], wr_ref[...], (((1,), (1,)), ((), ())),
                          preferred_element_type=jnp.float32)
  a = a_ref[...]
  out_ref[...] = jnp.maximum(h, 0.0) + a * jnp.minimum(h, 0.0)


def _dense_body(agg_ref, cnt_ref, x_ref, wl_ref, b_ref, wr_ref, out_ref):
  mean = (agg_ref[0, :N, :] + agg_ref[1, :N, :]) / cnt_ref[...]
  h = lax.dot_general(mean, wl_ref[...], (((1,), (1,)), ((), ())),
                      preferred_element_type=jnp.float32)
  h = h + b_ref[...]
  h = h + lax.dot_general(x_ref[...], wr_ref[...], (((1,), (1,)), ((), ())),
                          preferred_element_type=jnp.float32)
  out_ref[...] = h


_dense1 = pl.pallas_call(
    _dense_body_prelu,
    out_shape=jax.ShapeDtypeStruct((N, D), jnp.float32))
_dense2 = pl.pallas_call(
    _dense_body,
    out_shape=jax.ShapeDtypeStruct((N, D), jnp.float32))


@jax.jit
def kernel(x, edge_index, W1l, b1, W1r, a1, W2l, b2, W2r):
  src = edge_index[0]
  dst = edge_index[1]
  pad = E_PAD - E
  srcf = jnp.concatenate([src, jnp.zeros((pad,), jnp.int32)])
  dpad = DUMMY + jnp.arange(pad, dtype=jnp.int32) % (NPAD - N)
  dstf = jnp.concatenate([dst, dpad])
  srcp = srcf.reshape(NBLK, KB, CH)
  dstp = dstf.reshape(NBLK, KB, CH)

  cnt32 = _count(dstf.reshape(NW, K, CH))
  cntc = _cntsum(cnt32).reshape(NPAD, 1)[:N]
  b1r = b1.reshape(1, D)
  b2r = b2.reshape(1, D)

  (aggp,) = _segsum(x, srcp, dstp)
  h = _dense1(aggp, cntc, x, W1l, b1r, W1r, a1.reshape(1, 1))
  (aggp2,) = _segsum(h, srcp, dstp)
  out = _dense2(aggp2, cntc, h, W2l, b2r, W2r)
  return out


# speedup vs baseline: 1.0808x; 1.0200x over previous
import jax
import jax.numpy as jnp
from jax import lax
from jax.experimental import pallas as pl
from jax.experimental.pallas import tpu as pltpu
from jax.experimental.pallas import tpu_sc as plsc

N = 10000
E = 320000
D = 128

NC = 2
NS = 16
NW = NC * NS
CH = 128
KB = 8
NDEP = 2
EPW = 10240
K = EPW // CH
E_PAD = NW * EPW
NBLK = E_PAD // (KB * CH)
B0 = 17
B1 = 3
NPAD = 10240
RPT = NPAD // NS
DUMMY = N
LW = 128
NR = NPAD // LW
REG = 8

_mesh = plsc.VectorSubcoreMesh(core_axis_name="c", subcore_axis_name="s",
                               num_cores=NC, num_subcores=NS)


def _segsum_body(x_hbm, src_hbm, dst_hbm, agg_out, src_v, dst_v, b0, b1,
                 acc_sh, g0, g1, s0, s1):
  c = lax.axis_index("c")
  s = lax.axis_index("s")

  zv = jnp.zeros((16,), jnp.float32)

  def zrow(i, carry):
    for kk in range(D // 16):
      b0[i, pl.ds(kk * 16, 16)] = zv
    return carry

  lax.fori_loop(0, CH, zrow, 0)
  for r in range(RPT // CH):
    pltpu.sync_copy(b0, acc_sh.at[pl.ds(s * RPT + r * CH, CH)])
  plsc.subcore_barrier()

  bufs = (b0, b1)
  gsems = (g0, g1)
  ssems = (s0, s1)
  nblk = B0 + c * (B1 - B0)
  base = (1 - c) * (s * B0) + c * (NS * B0 + s * B1)

  def blk(b, carry):
    pltpu.sync_copy(src_hbm.at[base + b], src_v)
    pltpu.sync_copy(dst_hbm.at[base + b], dst_v)
    gcps = [None] * KB
    scps = [None] * KB
    for t in range(KB + NDEP - 1):
      jj = t
      if jj < KB:
        slot = jj % NDEP
        if jj >= NDEP:
          scps[jj - NDEP].wait()
        gcps[jj] = pltpu.async_copy(x_hbm.at[src_v.at[jj]], bufs[slot],
                                    gsems[slot])
      j2 = t - (NDEP - 1)
      if 0 <= j2 < KB:
        gcps[j2].wait()
        scps[j2] = pltpu.async_copy(bufs[j2 % NDEP],
                                    acc_sh.at[dst_v.at[j2]],
                                    ssems[j2 % NDEP], add=True)
    for j2 in range(KB - NDEP, KB):
      scps[j2].wait()
    return carry

  lax.fori_loop(0, nblk, blk, 0)
  plsc.subcore_barrier()

  pltpu.sync_copy(acc_sh.at[pl.ds(s * RPT, RPT)],
                  agg_out.at[c, pl.ds(s * RPT, RPT)])


_segsum = pl.kernel(
    _segsum_body,
    out_type=[jax.ShapeDtypeStruct((NC, NPAD, D), jnp.float32)],
    mesh=_mesh,
    scratch_types=[
        pltpu.VMEM((KB, CH), jnp.int32),
        pltpu.VMEM((KB, CH), jnp.int32),
        pltpu.VMEM((CH, D), jnp.float32),
        pltpu.VMEM((CH, D), jnp.float32),
        pltpu.VMEM_SHARED((NPAD, D), jnp.float32),
        pltpu.SemaphoreType.DMA,
        pltpu.SemaphoreType.DMA,
        pltpu.SemaphoreType.DMA,
        pltpu.SemaphoreType.DMA,
    ])


def _count_body(dst_hbm, cnt_out, dst_v, hist):
  c = lax.axis_index("c")
  s = lax.axis_index("s")
  w = s * NC + c
  pltpu.sync_copy(dst_hbm.at[w], dst_v)
  zv = jnp.zeros((16,), jnp.float32)

  def zrow(i, carry):
    for kk in range(LW // 16):
      hist[i, pl.ds(kk * 16, 16)] = zv
    return carry

  lax.fori_loop(0, REG * NR, zrow, 0)

  lane = lax.iota(jnp.int32, 16)
  region = jnp.bitwise_and(lane, REG - 1)
  mlo = lane < 8
  mhi = lane >= 8
  onesv = jnp.ones((16,), jnp.float32)

  def jrow(j, carry):
    for kk in range(CH // 16):
      d = dst_v[j, pl.ds(kk * 16, 16)]
      row = region * NR + lax.shift_right_logical(d, 7)
      col = jnp.bitwise_and(d, 127)
      plsc.addupdate_scatter(hist, [row, col], onesv, mask=mlo)
      plsc.addupdate_scatter(hist, [row, col], onesv, mask=mhi)
    return carry

  lax.fori_loop(0, K, jrow, 0)

  def rrow(i, carry):
    for kk in range(LW // 16):
      acc = hist[i, pl.ds(kk * 16, 16)]
      for r in range(1, REG):
        acc = acc + hist[r * NR + i, pl.ds(kk * 16, 16)]
      hist[i, pl.ds(kk * 16, 16)] = acc
    return carry

  lax.fori_loop(0, NR, rrow, 0)
  pltpu.sync_copy(hist.at[pl.ds(0, NR)], cnt_out.at[w])


_count = pl.kernel(
    _count_body,
    out_type=jax.ShapeDtypeStruct((NW, NR, LW), jnp.float32),
    mesh=_mesh,
    scratch_types=[pltpu.VMEM((K, CH), jnp.int32),
                   pltpu.VMEM((REG * NR, LW), jnp.float32)],
    compiler_params=pltpu.CompilerParams(needs_layout_passes=False))


def _cntsum_body(cnt_ref, out_ref):
  out_ref[...] = jnp.maximum(jnp.sum(cnt_ref[...], axis=0), 1.0)


_cntsum = pl.pallas_call(
    _cntsum_body,
    out_shape=jax.ShapeDtypeStruct((NR, LW), jnp.float32))


def _dense_body_prelu(agg_ref, cnt_ref, x_ref, wl_ref, b_ref, wr_ref,
                      a_ref, out_ref):
  mean = (agg_ref[0, :N, :] + agg_ref[1, :N, :]) / cnt_ref[...]
  h = lax.dot_general(mean, wl_ref[...], (((1,), (1,)), ((), ())),
                      preferred_element_type=jnp.float32)
  h = h + b_ref[...]
  h = h + lax.dot_general(x_ref[...], wr_ref[...], (((1,), (1,)), ((), ())),
                          preferred_element_type=jnp.float32)
  a = a_ref[...]
  out_ref[...] = jnp.maximum(h, 0.0) + a * jnp.minimum(h, 0.0)


def _dense_body(agg_ref, cnt_ref, x_ref, wl_ref, b_ref, wr_ref, out_ref):
  mean = (agg_ref[0, :N, :] + agg_ref[1, :N, :]) / cnt_ref[...]
  h = lax.dot_general(mean, wl_ref[...], (((1,), (1,)), ((), ())),
                      preferred_element_type=jnp.float32)
  h = h + b_ref[...]
  h = h + lax.dot_general(x_ref[...], wr_ref[...], (((1,), (1,)), ((), ())),
                          preferred_element_type=jnp.float32)
  out_ref[...] = h


_dense1 = pl.pallas_call(
    _dense_body_prelu,
    out_shape=jax.ShapeDtypeStruct((N, D), jnp.float32))
_dense2 = pl.pallas_call(
    _dense_body,
    out_shape=jax.ShapeDtypeStruct((N, D), jnp.float32))


@jax.jit
def kernel(x, edge_index, W1l, b1, W1r, a1, W2l, b2, W2r):
  src = edge_index[0]
  dst = edge_index[1]
  pad = E_PAD - E
  srcf = jnp.concatenate([src, jnp.zeros((pad,), jnp.int32)])
  dpad = DUMMY + jnp.arange(pad, dtype=jnp.int32) % (NPAD - N)
  dstf = jnp.concatenate([dst, dpad])
  srcp = srcf.reshape(NBLK, KB, CH)
  dstp = dstf.reshape(NBLK, KB, CH)

  cnt32 = _count(dstf.reshape(NW, K, CH))
  cntc = _cntsum(cnt32).reshape(NPAD, 1)[:N]
  b1r = b1.reshape(1, D)
  b2r = b2.reshape(1, D)

  (aggp,) = _segsum(x, srcp, dstp)
  h = _dense1(aggp, cntc, x, W1l, b1r, W1r, a1.reshape(1, 1))
  (aggp2,) = _segsum(h, srcp, dstp)
  out = _dense2(aggp2, cntc, h, W2l, b2r, W2r)
  return out


# speedup vs baseline: 1.1807x; 1.0925x over previous
import jax
import jax.numpy as jnp
from jax import lax
from jax.experimental import pallas as pl
from jax.experimental.pallas import tpu as pltpu
from jax.experimental.pallas import tpu_sc as plsc

N = 10000
E = 320000
D = 128

NC = 2
NS = 16
NW = NC * NS
CH = 128
KB = 8
NDEP = 2
EPW = 10240
K = EPW // CH
E_PAD = NW * EPW
NBLK = E_PAD // (KB * CH)
B0 = 18
B1 = 2
NPAD = 10240
RPT = NPAD // NS
DUMMY = N
LW = 128
NR = NPAD // LW
REG = 8

_mesh = plsc.VectorSubcoreMesh(core_axis_name="c", subcore_axis_name="s",
                               num_cores=NC, num_subcores=NS)


def _segsum_body(x_hbm, src_hbm, dst_hbm, agg_out, src_v, dst_v, b0, b1,
                 acc_sh, g0, g1, s0, s1):
  c = lax.axis_index("c")
  s = lax.axis_index("s")

  zv = jnp.zeros((16,), jnp.float32)

  def zrow(i, carry):
    for kk in range(D // 16):
      b0[i, pl.ds(kk * 16, 16)] = zv
    return carry

  lax.fori_loop(0, CH, zrow, 0)
  for r in range(RPT // CH):
    pltpu.sync_copy(b0, acc_sh.at[pl.ds(s * RPT + r * CH, CH)])
  plsc.subcore_barrier()

  bufs = (b0, b1)
  gsems = (g0, g1)
  ssems = (s0, s1)
  nblk = B0 + c * (B1 - B0)
  base = (1 - c) * (s * B0) + c * (NS * B0 + s * B1)

  def blk(b, carry):
    pltpu.sync_copy(src_hbm.at[base + b], src_v)
    pltpu.sync_copy(dst_hbm.at[base + b], dst_v)
    gcps = [None] * KB
    scps = [None] * KB
    for t in range(KB + NDEP - 1):
      jj = t
      if jj < KB:
        slot = jj % NDEP
        if jj >= NDEP:
          scps[jj - NDEP].wait()
        gcps[jj] = pltpu.async_copy(x_hbm.at[src_v.at[jj]], bufs[slot],
                                    gsems[slot])
      j2 = t - (NDEP - 1)
      if 0 <= j2 < KB:
        gcps[j2].wait()
        scps[j2] = pltpu.async_copy(bufs[j2 % NDEP],
                                    acc_sh.at[dst_v.at[j2]],
                                    ssems[j2 % NDEP], add=True)
    for j2 in range(KB - NDEP, KB):
      scps[j2].wait()
    return carry

  lax.fori_loop(0, nblk, blk, 0)
  plsc.subcore_barrier()

  pltpu.sync_copy(acc_sh.at[pl.ds(s * RPT, RPT)],
                  agg_out.at[c, pl.ds(s * RPT, RPT)])


_segsum = pl.kernel(
    _segsum_body,
    out_type=[jax.ShapeDtypeStruct((NC, NPAD, D), jnp.float32)],
    mesh=_mesh,
    scratch_types=[
        pltpu.VMEM((KB, CH), jnp.int32),
        pltpu.VMEM((KB, CH), jnp.int32),
        pltpu.VMEM((CH, D), jnp.float32),
        pltpu.VMEM((CH, D), jnp.float32),
        pltpu.VMEM_SHARED((NPAD, D), jnp.float32),
        pltpu.SemaphoreType.DMA,
        pltpu.SemaphoreType.DMA,
        pltpu.SemaphoreType.DMA,
        pltpu.SemaphoreType.DMA,
    ])


def _count_body(dst_hbm, cnt_out, dst_v, hist):
  c = lax.axis_index("c")
  s = lax.axis_index("s")
  w = s * NC + c
  pltpu.sync_copy(dst_hbm.at[w], dst_v)
  zv = jnp.zeros((16,), jnp.float32)

  def zrow(i, carry):
    for kk in range(LW // 16):
      hist[i, pl.ds(kk * 16, 16)] = zv
    return carry

  lax.fori_loop(0, REG * NR, zrow, 0)

  lane = lax.iota(jnp.int32, 16)
  region = jnp.bitwise_and(lane, REG - 1)
  mlo = lane < 8
  mhi = lane >= 8
  onesv = jnp.ones((16,), jnp.float32)

  def jrow(j, carry):
    for kk in range(CH // 16):
      d = dst_v[j, pl.ds(kk * 16, 16)]
      row = region * NR + lax.shift_right_logical(d, 7)
      col = jnp.bitwise_and(d, 127)
      plsc.addupdate_scatter(hist, [row, col], onesv, mask=mlo)
      plsc.addupdate_scatter(hist, [row, col], onesv, mask=mhi)
    return carry

  lax.fori_loop(0, K, jrow, 0)

  def rrow(i, carry):
    for kk in range(LW // 16):
      acc = hist[i, pl.ds(kk * 16, 16)]
      for r in range(1, REG):
        acc = acc + hist[r * NR + i, pl.ds(kk * 16, 16)]
      hist[i, pl.ds(kk * 16, 16)] = acc
    return carry

  lax.fori_loop(0, NR, rrow, 0)
  pltpu.sync_copy(hist.at[pl.ds(0, NR)], cnt_out.at[w])


_count = pl.kernel(
    _count_body,
    out_type=jax.ShapeDtypeStruct((NW, NR, LW), jnp.float32),
    mesh=_mesh,
    scratch_types=[pltpu.VMEM((K, CH), jnp.int32),
                   pltpu.VMEM((REG * NR, LW), jnp.float32)],
    compiler_params=pltpu.CompilerParams(needs_layout_passes=False))


def _cntsum_body(cnt_ref, out_ref):
  out_ref[...] = jnp.maximum(jnp.sum(cnt_ref[...], axis=0), 1.0)


_cntsum = pl.pallas_call(
    _cntsum_body,
    out_shape=jax.ShapeDtypeStruct((NR, LW), jnp.float32))


def _dense_body_prelu(agg_ref, cnt_ref, x_ref, wl_ref, b_ref, wr_ref,
                      a_ref, out_ref):
  mean = (agg_ref[0, :N, :] + agg_ref[1, :N, :]) / cnt_ref[...]
  h = lax.dot_general(mean, wl_ref[...], (((1,), (1,)), ((), ())),
                      preferred_element_type=jnp.float32)
  h = h + b_ref[...]
  h = h + lax.dot_general(x_ref[...], wr_ref[...], (((1,), (1,)), ((), ())),
                          preferred_element_type=jnp.float32)
  a = a_ref[...]
  out_ref[...] = jnp.maximum(h, 0.0) + a * jnp.minimum(h, 0.0)


def _dense_body(agg_ref, cnt_ref, x_ref, wl_ref, b_ref, wr_ref, out_ref):
  mean = (agg_ref[0, :N, :] + agg_ref[1, :N, :]) / cnt_ref[...]
  h = lax.dot_general(mean, wl_ref[...], (((1,), (1,)), ((), ())),
                      preferred_element_type=jnp.float32)
  h = h + b_ref[...]
  h = h + lax.dot_general(x_ref[...], wr_ref[...], (((1,), (1,)), ((), ())),
                          preferred_element_type=jnp.float32)
  out_ref[...] = h


_dense1 = pl.pallas_call(
    _dense_body_prelu,
    out_shape=jax.ShapeDtypeStruct((N, D), jnp.float32))
_dense2 = pl.pallas_call(
    _dense_body,
    out_shape=jax.ShapeDtypeStruct((N, D), jnp.float32))


@jax.jit
def kernel(x, edge_index, W1l, b1, W1r, a1, W2l, b2, W2r):
  src = edge_index[0]
  dst = edge_index[1]
  pad = E_PAD - E
  srcf = jnp.concatenate([src, jnp.zeros((pad,), jnp.int32)])
  dpad = DUMMY + jnp.arange(pad, dtype=jnp.int32) % (NPAD - N)
  dstf = jnp.concatenate([dst, dpad])
  srcp = srcf.reshape(NBLK, KB, CH)
  dstp = dstf.reshape(NBLK, KB, CH)

  cnt32 = _count(dstf.reshape(NW, K, CH))
  cntc = _cntsum(cnt32).reshape(NPAD, 1)[:N]
  b1r = b1.reshape(1, D)
  b2r = b2.reshape(1, D)

  (aggp,) = _segsum(x, srcp, dstp)
  h = _dense1(aggp, cntc, x, W1l, b1r, W1r, a1.reshape(1, 1))
  (aggp2,) = _segsum(h, srcp, dstp)
  out = _dense2(aggp2, cntc, h, W2l, b2r, W2r)
  return out
